# Initial kernel scaffold; baseline (speedup 1.0000x reference)
import jax
import jax.numpy as jnp
from jax.experimental import pallas as pl


def kernel(x, table):
    raise NotImplementedError("write your pallas kernel here")


# speedup vs baseline: 1.0945x; 1.0945x over previous
import functools

import jax
import jax.numpy as jnp
from jax import lax
from jax.experimental import pallas as pl
from jax.experimental.pallas import tpu as pltpu
from jax.experimental.pallas import tpu_sc as plsc

_NUM_CORES = 2
_NUM_SUBCORES = 16
_NUM_WORKERS = _NUM_CORES * _NUM_SUBCORES

_CHUNK = 1024


def _emb_lookup(flat_idx, table, *, chunk):
    B, = flat_idx.shape
    V, D = table.shape
    b_per_w = B // _NUM_WORKERS
    n_chunks = b_per_w // chunk

    mesh = plsc.VectorSubcoreMesh(core_axis_name="c", subcore_axis_name="s")

    @functools.partial(
        pl.kernel,
        out_type=jax.ShapeDtypeStruct((B, D), jnp.float32),
        mesh=mesh,
        compiler_params=pltpu.CompilerParams(use_tc_tiling_on_sc=False),
        scratch_types=[
            pltpu.VMEM((chunk,), jnp.int32),
            pltpu.VMEM((chunk, D), jnp.float32),
            pltpu.SemaphoreType.DMA,
        ],
    )
    def emb(idx_hbm, table_hbm, out_hbm, idx_v, rows_v, sem):
        wid = lax.axis_index("s") * _NUM_CORES + lax.axis_index("c")
        base = wid * b_per_w

        def body(i, carry):
            off = base + i * chunk
            pltpu.sync_copy(idx_hbm.at[pl.ds(off, chunk)], idx_v)
            pltpu.async_copy(table_hbm.at[idx_v], rows_v, sem).wait()
            pltpu.sync_copy(rows_v, out_hbm.at[pl.ds(off, chunk)])
            return carry

        lax.fori_loop(0, n_chunks, body, 0)

    return emb(flat_idx, table)


def kernel(x, table):
    Bx, H = x.shape
    V, D = table.shape
    flat_idx = x.reshape(Bx * H).astype(jnp.int32)
    out = _emb_lookup(flat_idx, table, chunk=_CHUNK)
    return out.reshape(Bx, H, D)


# speedup vs baseline: 1.1138x; 1.0176x over previous
import functools

import jax
import jax.numpy as jnp
from jax import lax
from jax.experimental import pallas as pl
from jax.experimental.pallas import tpu as pltpu
from jax.experimental.pallas import tpu_sc as plsc

_NUM_CORES = 2
_NUM_SUBCORES = 16
_NUM_WORKERS = _NUM_CORES * _NUM_SUBCORES

_CHUNK = 1280


def _emb_lookup(idx2d, table, *, chunk):
    n_rows, chunk_ = idx2d.shape
    assert chunk_ == chunk
    V, D = table.shape
    n_chunks = n_rows // _NUM_WORKERS
    B = n_rows * chunk

    mesh = plsc.VectorSubcoreMesh(core_axis_name="c", subcore_axis_name="s")

    @functools.partial(
        pl.kernel,
        out_type=jax.ShapeDtypeStruct((B, D), jnp.float32),
        mesh=mesh,
        compiler_params=pltpu.CompilerParams(use_tc_tiling_on_sc=False),
        scratch_types=[
            pltpu.VMEM((n_chunks, chunk), jnp.int32),
            pltpu.VMEM((2, chunk, D), jnp.float32),
            pltpu.SemaphoreType.DMA((2,)),
            pltpu.SemaphoreType.DMA((2,)),
            pltpu.SemaphoreType.DMA,
        ],
    )
    def emb(idx_hbm, table_hbm, out_hbm, idx_v, rows_v, gsem, osem, isem):
        wid = lax.axis_index("s") * _NUM_CORES + lax.axis_index("c")
        base = wid * n_chunks

        pltpu.async_copy(idx_hbm.at[pl.ds(base, n_chunks)], idx_v, isem).wait()

        def gather(i, b):
            return pltpu.async_copy(
                table_hbm.at[idx_v.at[i]], rows_v.at[b], gsem.at[b])

        def copy_out(i, b):
            return pltpu.make_async_copy(
                rows_v.at[b],
                out_hbm.at[pl.ds((base + i) * chunk, chunk)],
                osem.at[b])

        gather(0, 0)

        def body(i, carry):
            b = lax.rem(i, 2)
            nb = lax.rem(i + 1, 2)

            @pl.when(i + 1 < n_chunks)
            def _start_next():
                @pl.when(i >= 1)
                def _drain_prev_out():
                    copy_out(i - 1, nb).wait()
                gather(i + 1, nb)

            pltpu.make_async_copy(
                table_hbm.at[idx_v.at[i]], rows_v.at[b], gsem.at[b]).wait()
            copy_out(i, b).start()
            return carry

        lax.fori_loop(0, n_chunks, body, 0)

        copy_out(n_chunks - 2, lax.rem(n_chunks - 2, 2)).wait()
        copy_out(n_chunks - 1, lax.rem(n_chunks - 1, 2)).wait()

    return emb(idx2d, table)


def kernel(x, table):
    Bx, H = x.shape
    V, D = table.shape
    B = Bx * H
    n_rows = B // _CHUNK
    idx2d = x.reshape(n_rows, _CHUNK).astype(jnp.int32)
    out = _emb_lookup(idx2d, table, chunk=_CHUNK)
    return out.reshape(Bx, H, D)
